# baseline (speedup 1.0000x reference)
import functools

import jax
import jax.numpy as jnp
from jax import lax
from jax.experimental import pallas as pl
from jax.experimental.pallas import tpu as pltpu
from jax.experimental.pallas import tpu_sc as plsc

B = 16384
F_CAT = 26
F_REAL = 13
V = 100000
D = 32
F_TOT = F_CAT + F_REAL

NC = 2
NS = 16
NW = NC * NS
BPW = B // NW
GCH = 128
NCH = BPW // GCH


def _sc_gather_body(idx_hbm, tab_hbm, out_hbm, idx_v, rows_v, sem):
    c = lax.axis_index("c")
    s = lax.axis_index("s")
    wid = s * NC + c
    base = wid * BPW
    for f in range(F_CAT):
        pltpu.sync_copy(idx_hbm.at[f, pl.ds(base, BPW)], idx_v)
        descs = []
        for ch in range(NCH):
            descs.append(
                pltpu.async_copy(
                    tab_hbm.at[idx_v.at[pl.ds(ch * GCH, GCH)]],
                    rows_v.at[pl.ds(ch * GCH, GCH)],
                    sem,
                )
            )
        for d in descs:
            d.wait()
        pltpu.sync_copy(rows_v, out_hbm.at[pl.ds(base, BPW), f])


_sc_gather = functools.partial(
    pl.kernel,
    out_type=jax.ShapeDtypeStruct((B, F_TOT, D), jnp.float32),
    mesh=plsc.VectorSubcoreMesh(core_axis_name="c", subcore_axis_name="s"),
    scratch_types=[
        pltpu.VMEM((BPW,), jnp.int32),
        pltpu.VMEM((BPW, D), jnp.float32),
        pltpu.SemaphoreType.DMA,
    ],
    compiler_params=pltpu.CompilerParams(use_tc_tiling_on_sc=False),
)(_sc_gather_body)


def _tc_sweep_body(x_ref, w_ref, b_ref, g_ref, be_ref, t_ref, io_ref, out_ref):
    cat = io_ref[:, :F_CAT, :] + t_ref[...][None, :F_CAT, :]
    x = x_ref[...]
    w = w_ref[...]
    b = b_ref[...]
    h = x[:, :, None] * w[None] + b[None]
    mu = jnp.mean(h, axis=-1, keepdims=True)
    var = jnp.mean((h - mu) * (h - mu), axis=-1, keepdims=True)
    h = (h - mu) * lax.rsqrt(var + 1e-5)
    h = h * g_ref[...][None] + be_ref[...][None]
    h = jnp.maximum(h, 0.0)
    real = h + t_ref[...][None, F_CAT:, :]
    out_ref[...] = jnp.concatenate([cat, real], axis=1)


BBLK = 512


def _tc_sweep(ur, real_w, real_b, ln_gamma, ln_beta, type_emb, combined):
    return pl.pallas_call(
        _tc_sweep_body,
        out_shape=jax.ShapeDtypeStruct((B, F_TOT, D), jnp.float32),
        grid=(B // BBLK,),
        in_specs=[
            pl.BlockSpec((BBLK, F_REAL), lambda i: (i, 0)),
            pl.BlockSpec((F_REAL, D), lambda i: (0, 0)),
            pl.BlockSpec((F_REAL, D), lambda i: (0, 0)),
            pl.BlockSpec((F_REAL, D), lambda i: (0, 0)),
            pl.BlockSpec((F_REAL, D), lambda i: (0, 0)),
            pl.BlockSpec((F_TOT, D), lambda i: (0, 0)),
            pl.BlockSpec((BBLK, F_TOT, D), lambda i: (i, 0, 0)),
        ],
        out_specs=pl.BlockSpec((BBLK, F_TOT, D), lambda i: (i, 0, 0)),
        input_output_aliases={6: 0},
    )(ur, real_w, real_b, ln_gamma, ln_beta, type_emb, combined)


def kernel(user_categoricals, user_reals, cat_tables, type_emb, real_w, real_b,
           ln_gamma, ln_beta):
    offs = (jnp.arange(F_CAT, dtype=jnp.int32) * V)[:, None]
    idx_t = user_categoricals.T + offs
    flat_tab = cat_tables.reshape(F_CAT * V, D)
    combined = _sc_gather(idx_t, flat_tab)
    return _tc_sweep(user_reals, real_w, real_b, ln_gamma, ln_beta,
                     type_emb, combined)


# speedup vs baseline: 1.0102x; 1.0102x over previous
import functools

import jax
import jax.numpy as jnp
from jax import lax
from jax.experimental import pallas as pl
from jax.experimental.pallas import tpu as pltpu
from jax.experimental.pallas import tpu_sc as plsc

B = 16384
F_CAT = 26
F_REAL = 13
V = 100000
D = 32
F_TOT = F_CAT + F_REAL

NC = 2
NS = 16
NW = NC * NS
BPW = B // NW
GCH = 128
NCH = BPW // GCH


def _sc_gather_body(uc_hbm, tab_hbm, out_hbm, idx2_v, cols_v, buf_v, gsem, osem):
    c = lax.axis_index("c")
    s = lax.axis_index("s")
    wid = s * NC + c
    base = wid * BPW
    pltpu.sync_copy(uc_hbm.at[pl.ds(base, BPW)], idx2_v)

    lanes = lax.broadcasted_iota(jnp.int32, (16,), 0)
    zeros = jnp.zeros((16,), jnp.int32)

    def ext_body(t, _):
        f = t // (BPW // 16)
        j = t % (BPW // 16)
        rows = lanes + j * 16
        fcol = zeros + f
        v = plsc.load_gather(idx2_v, [rows, fcol])
        cols_v[f, pl.ds(j * 16, 16)] = v
        return 0

    lax.fori_loop(0, F_CAT * (BPW // 16), ext_body, 0)

    def gathers(f):
        ds = []
        for ch in range(NCH):
            ds.append(
                pltpu.async_copy(
                    tab_hbm.at[f].at[cols_v.at[f, pl.ds(ch * GCH, GCH)]],
                    buf_v.at[f % 2].at[pl.ds(ch * GCH, GCH)],
                    gsem,
                )
            )
        return ds

    gd = {}
    od = {}
    for f in range(F_CAT):
        if f >= 2:
            od[f - 2].wait()
        gd[f] = gathers(f)
        if f >= 1:
            for d in gd[f - 1]:
                d.wait()
            od[f - 1] = pltpu.async_copy(
                buf_v.at[(f - 1) % 2], out_hbm.at[pl.ds(base, BPW), f - 1], osem
            )
    for d in gd[F_CAT - 1]:
        d.wait()
    od[F_CAT - 1] = pltpu.async_copy(
        buf_v.at[(F_CAT - 1) % 2], out_hbm.at[pl.ds(base, BPW), F_CAT - 1], osem
    )
    od[F_CAT - 2].wait()
    od[F_CAT - 1].wait()


_sc_gather = functools.partial(
    pl.kernel,
    out_type=jax.ShapeDtypeStruct((B, F_TOT, D), jnp.float32),
    mesh=plsc.VectorSubcoreMesh(core_axis_name="c", subcore_axis_name="s"),
    scratch_types=[
        pltpu.VMEM((BPW, F_CAT), jnp.int32),
        pltpu.VMEM((F_CAT, BPW), jnp.int32),
        pltpu.VMEM((2, BPW, D), jnp.float32),
        pltpu.SemaphoreType.DMA,
        pltpu.SemaphoreType.DMA,
    ],
    compiler_params=pltpu.CompilerParams(
        use_tc_tiling_on_sc=False, needs_layout_passes=False
    ),
)(_sc_gather_body)


def _tc_sweep_body(x_ref, w_ref, b_ref, g_ref, be_ref, t_ref, io_ref, out_ref):
    cat = io_ref[:, :F_CAT, :] + t_ref[...][None, :F_CAT, :]
    x = x_ref[...]
    w = w_ref[...]
    b = b_ref[...]
    h = x[:, :, None] * w[None] + b[None]
    mu = jnp.mean(h, axis=-1, keepdims=True)
    var = jnp.mean((h - mu) * (h - mu), axis=-1, keepdims=True)
    h = (h - mu) * lax.rsqrt(var + 1e-5)
    h = h * g_ref[...][None] + be_ref[...][None]
    h = jnp.maximum(h, 0.0)
    real = h + t_ref[...][None, F_CAT:, :]
    out_ref[...] = jnp.concatenate([cat, real], axis=1)


BBLK = 512


def _tc_sweep(ur, real_w, real_b, ln_gamma, ln_beta, type_emb, combined):
    return pl.pallas_call(
        _tc_sweep_body,
        out_shape=jax.ShapeDtypeStruct((B, F_TOT, D), jnp.float32),
        grid=(B // BBLK,),
        in_specs=[
            pl.BlockSpec((BBLK, F_REAL), lambda i: (i, 0)),
            pl.BlockSpec((F_REAL, D), lambda i: (0, 0)),
            pl.BlockSpec((F_REAL, D), lambda i: (0, 0)),
            pl.BlockSpec((F_REAL, D), lambda i: (0, 0)),
            pl.BlockSpec((F_REAL, D), lambda i: (0, 0)),
            pl.BlockSpec((F_TOT, D), lambda i: (0, 0)),
            pl.BlockSpec((BBLK, F_TOT, D), lambda i: (i, 0, 0)),
        ],
        out_specs=pl.BlockSpec((BBLK, F_TOT, D), lambda i: (i, 0, 0)),
        input_output_aliases={6: 0},
    )(ur, real_w, real_b, ln_gamma, ln_beta, type_emb, combined)


def kernel(user_categoricals, user_reals, cat_tables, type_emb, real_w, real_b,
           ln_gamma, ln_beta):
    combined = _sc_gather(user_categoricals, cat_tables)
    return _tc_sweep(user_reals, real_w, real_b, ln_gamma, ln_beta,
                     type_emb, combined)
